# Initial kernel scaffold; baseline (speedup 1.0000x reference)
import jax
import jax.numpy as jnp
from jax.experimental import pallas as pl


def kernel(loc, conf, landm, dpt, boxes_gt, keypoints_gt, depths_gt, priors, labels_gt):
    raise NotImplementedError("write your pallas kernel here")


# speedup vs baseline: 49.0211x; 49.0211x over previous
import jax
import jax.numpy as jnp
from jax import lax
from jax.experimental import pallas as pl

_NUM_CLASSES = 2
_THRESHOLD = 0.35
_NEGPOS = 7
_V0, _V1 = 0.1, 0.2
_B, _P, _NOBJ = 64, 16800, 32


def _mbl_kernel(loc_ref, conf_ref, landm_ref, dpt_ref,
                boxes_ref, boxes_t_ref, keyp_t_ref, dep_t_ref, lab_ref,
                priors_t_ref, out_ref):
    b = pl.program_id(0)

    pr = priors_t_ref[...]
    pcx, pcy, pw, ph = pr[0:1], pr[1:2], pr[2:3], pr[3:4]
    px1 = pcx - pw / 2.0
    py1 = pcy - ph / 2.0
    px2 = pcx + pw / 2.0
    py2 = pcy + ph / 2.0
    area_p = (px2 - px1) * (py2 - py1)

    bg = boxes_ref[0]
    gx1, gy1, gx2, gy2 = bg[:, 0:1], bg[:, 1:2], bg[:, 2:3], bg[:, 3:4]
    area_g = (gx2 - gx1) * (gy2 - gy1)

    ix1 = jnp.maximum(gx1, px1)
    iy1 = jnp.maximum(gy1, py1)
    ix2 = jnp.minimum(gx2, px2)
    iy2 = jnp.minimum(gy2, py2)
    iw = jnp.maximum(ix2 - ix1, 0.0)
    ih = jnp.maximum(iy2 - iy1, 0.0)
    inter = iw * ih
    overlaps = inter / (area_g + area_p - inter)

    idx_l = lax.broadcasted_iota(jnp.int32, (_NOBJ, _P), 1)
    idx_s = lax.broadcasted_iota(jnp.int32, (_NOBJ, _P), 0)

    rowmax = jnp.max(overlaps, axis=1, keepdims=True)
    best_prior = jnp.min(jnp.where(overlaps == rowmax, idx_l, _P),
                         axis=1, keepdims=True)

    colmax = jnp.max(overlaps, axis=0, keepdims=True)
    bt_idx = jnp.min(jnp.where(overlaps == colmax, idx_s, _NOBJ),
                     axis=0, keepdims=True)

    claim = idx_l == best_prior
    claimed = jnp.max(jnp.where(claim, 1, 0), axis=0, keepdims=True) > 0
    claim_j = jnp.max(jnp.where(claim, idx_s, -1), axis=0, keepdims=True)
    idx2 = jnp.where(claimed, claim_j, bt_idx)
    ovl2 = jnp.where(claimed, 2.0, colmax)

    onehot = (idx_s == idx2).astype(jnp.float32)
    dn = (((1,), (0,)), ((), ()))
    lab = lab_ref[0]
    labels_sel = lax.dot_general(lab, onehot, dn,
                                 preferred_element_type=jnp.float32)
    labels_v = jnp.where(ovl2 < _THRESHOLD, 0.0, labels_sel)
    positive = labels_v > 0.0
    posf = positive.astype(jnp.float32)
    num_pos = jnp.sum(positive.astype(jnp.int32))

    bsel = lax.dot_general(boxes_t_ref[0], onehot, dn,
                           preferred_element_type=jnp.float32)
    ksel = lax.dot_general(keyp_t_ref[0], onehot, dn,
                           preferred_element_type=jnp.float32)
    dsel = lax.dot_general(dep_t_ref[0], onehot, dn,
                           preferred_element_type=jnp.float32)

    mx1, my1, mx2, my2 = bsel[0:1], bsel[1:2], bsel[2:3], bsel[3:4]
    gcx = ((mx1 + mx2) / 2.0 - pcx) / (_V0 * pw)
    gcy = ((my1 + my2) / 2.0 - pcy) / (_V0 * ph)
    gw = jnp.log(jnp.maximum((mx2 - mx1) / pw, 1e-8)) / _V1
    gh = jnp.log(jnp.maximum((my2 - my1) / ph, 1e-8)) / _V1
    benc = jnp.concatenate([gcx, gcy, gw, gh], axis=0)

    def smooth_l1(x, y):
        d = jnp.abs(x - y)
        return jnp.where(d < 1.0, 0.5 * d * d, d - 0.5)

    loc_t = loc_ref[0]
    loss_l = jnp.sum(smooth_l1(loc_t, benc) * posf)

    pc_rep = jnp.concatenate([pcx, pcy] * 5, axis=0)
    pd_rep = jnp.concatenate([_V0 * pw, _V0 * ph] * 5, axis=0)
    kenc = (ksel - pc_rep) / pd_rep
    loss_landm = jnp.sum(smooth_l1(kenc, landm_ref[0]) * posf)

    loss_dpth = jnp.sum(((dsel - dpt_ref[0]) ** 2) * posf)

    cf = conf_ref[0]
    c0, c1 = cf[0:1], cf[1:2]
    cm = jnp.maximum(c0, c1)
    lse = cm + jnp.log(jnp.exp(c0 - cm) + jnp.exp(c1 - cm))
    ce = lse - jnp.where(positive, c1, c0)
    v = jnp.where(positive, 0.0, lse - c0)

    num_neg = jnp.minimum(_NEGPOS * num_pos, _P - 1)
    k = jnp.maximum(num_neg, 1)

    bits = lax.bitcast_convert_type(v, jnp.int32)
    iota_l = lax.broadcasted_iota(jnp.int32, (1, _P), 1)

    def vbody(_, carry):
        lo, hi = carry
        mid = lo + (hi - lo) // 2
        cnt = jnp.sum((bits >= mid).astype(jnp.int32))
        ge = cnt >= k
        return (jnp.where(ge, mid, lo), jnp.where(ge, hi, mid))

    lo, _hi = lax.fori_loop(0, 31, vbody,
                            (jnp.int32(0), jnp.int32(0x7F800001)))
    t = lo
    count_gt = jnp.sum((bits > t).astype(jnp.int32))
    need_eq = k - count_gt
    eq = bits == t

    def ibody(_, carry):
        lo2, hi2 = carry
        active = lo2 < hi2
        mid = (lo2 + hi2) // 2
        cnt = jnp.sum((eq & (iota_l < mid)).astype(jnp.int32))
        ge = cnt >= need_eq
        nlo = jnp.where(ge, lo2, mid + 1)
        nhi = jnp.where(ge, mid, hi2)
        return (jnp.where(active, nlo, lo2), jnp.where(active, nhi, hi2))

    _lo2, m_thr = lax.fori_loop(0, 15, ibody,
                                (jnp.int32(0), jnp.int32(_P)))

    neg_sel = (bits > t) | (eq & (iota_l < m_thr))
    sel = positive | (neg_sel & (num_neg > 0))
    loss_c = jnp.sum(jnp.where(sel, ce, 0.0))

    lane = lax.broadcasted_iota(jnp.int32, (1, 128), 1)
    vec = (jnp.where(lane == 0, loss_l, 0.0)
           + jnp.where(lane == 1, loss_c, 0.0)
           + jnp.where(lane == 2, loss_landm, 0.0)
           + jnp.where(lane == 3, loss_dpth, 0.0)
           + jnp.where(lane == 4, num_pos.astype(jnp.float32), 0.0))

    @pl.when(b == 0)
    def _():
        out_ref[...] = jnp.zeros_like(out_ref)

    out_ref[...] += vec


def kernel(loc, conf, landm, dpt, boxes_gt, keypoints_gt, depths_gt, priors, labels_gt):
    loc_t = jnp.transpose(loc, (0, 2, 1))
    conf_t = jnp.transpose(conf, (0, 2, 1))
    landm_t = jnp.transpose(landm, (0, 2, 1))
    dpt_t = jnp.transpose(dpt, (0, 2, 1))
    boxes_t = jnp.transpose(boxes_gt, (0, 2, 1))
    keyp_t = jnp.transpose(keypoints_gt, (0, 2, 1))
    dep_t = jnp.transpose(depths_gt, (0, 2, 1))
    lab_f = labels_gt.astype(jnp.float32)[:, None, :]
    priors_t = priors.T

    grid = (_B,)
    bs = lambda shp: pl.BlockSpec((1,) + shp, lambda b: (b, 0, 0))
    out = pl.pallas_call(
        _mbl_kernel,
        grid=grid,
        in_specs=[
            bs((4, _P)),
            bs((2, _P)),
            bs((10, _P)),
            bs((2, _P)),
            bs((_NOBJ, 4)),
            bs((4, _NOBJ)),
            bs((10, _NOBJ)),
            bs((2, _NOBJ)),
            bs((1, _NOBJ)),
            pl.BlockSpec((4, _P), lambda b: (0, 0)),
        ],
        out_specs=pl.BlockSpec((1, 128), lambda b: (0, 0)),
        out_shape=jax.ShapeDtypeStruct((1, 128), jnp.float32),
    )(loc_t, conf_t, landm_t, dpt_t,
      boxes_gt, boxes_t, keyp_t, dep_t, lab_f, priors_t)

    r = out[0]
    npos = r[4]
    n = jnp.maximum(npos, 1.0)
    n1 = jnp.maximum(npos * 10.0, 1.0)
    nd = jnp.maximum(npos * 2.0, 1.0)
    return (r[0] / n, r[1] / n, r[2] / n1, r[3] / nd)


# speedup vs baseline: 104.0435x; 2.1224x over previous
import jax
import jax.numpy as jnp
from jax import lax
from jax.experimental import pallas as pl

_NUM_CLASSES = 2
_THRESHOLD = 0.35
_NEGPOS = 7
_V0, _V1 = 0.1, 0.2
_B, _P, _NOBJ = 64, 16800, 32


_S = 8


def _mbl_kernel(loc_ref, conf_ref, landm_ref, dpt_ref,
                boxes_ref, boxes_t_ref, keyp_t_ref, dep_t_ref, lab_ref,
                priors_t_ref, out_ref):
    b = pl.program_id(0)

    pr = priors_t_ref[...]
    pcx, pcy, pw, ph = pr[0:1], pr[1:2], pr[2:3], pr[3:4]
    px1 = pcx - pw / 2.0
    py1 = pcy - ph / 2.0
    px2 = pcx + pw / 2.0
    py2 = pcy + ph / 2.0
    area_p = (px2 - px1) * (py2 - py1)

    idx_l = lax.broadcasted_iota(jnp.int32, (_NOBJ, _P), 1)
    idx_s = lax.broadcasted_iota(jnp.int32, (_NOBJ, _P), 0)
    dn = (((1,), (0,)), ((), ()))

    def smooth_l1(x, y):
        d = jnp.abs(x - y)
        return jnp.where(d < 1.0, 0.5 * d * d, d - 0.5)

    pc_rep = jnp.concatenate([pcx, pcy] * 5, axis=0)
    pd_rep = jnp.concatenate([_V0 * pw, _V0 * ph] * 5, axis=0)

    loss_l = jnp.float32(0.0)
    loss_landm = jnp.float32(0.0)
    loss_dpth = jnp.float32(0.0)
    pos_rows = []
    ce_rows = []
    v_rows = []

    for s in range(_S):
        bg = boxes_ref[s]
        gx1, gy1, gx2, gy2 = bg[:, 0:1], bg[:, 1:2], bg[:, 2:3], bg[:, 3:4]
        area_g = (gx2 - gx1) * (gy2 - gy1)

        ix1 = jnp.maximum(gx1, px1)
        iy1 = jnp.maximum(gy1, py1)
        ix2 = jnp.minimum(gx2, px2)
        iy2 = jnp.minimum(gy2, py2)
        iw = jnp.maximum(ix2 - ix1, 0.0)
        ih = jnp.maximum(iy2 - iy1, 0.0)
        inter = iw * ih
        overlaps = inter / (area_g + area_p - inter)

        rowmax = jnp.max(overlaps, axis=1, keepdims=True)
        best_prior = jnp.min(jnp.where(overlaps == rowmax, idx_l, _P),
                             axis=1, keepdims=True)

        colmax = jnp.max(overlaps, axis=0, keepdims=True)
        bt_idx = jnp.min(jnp.where(overlaps == colmax, idx_s, _NOBJ),
                         axis=0, keepdims=True)

        claim = idx_l == best_prior
        claimed = jnp.max(jnp.where(claim, 1, 0), axis=0,
                          keepdims=True) > 0
        claim_j = jnp.max(jnp.where(claim, idx_s, -1), axis=0,
                          keepdims=True)
        idx2 = jnp.where(claimed, claim_j, bt_idx)
        ovl2 = jnp.where(claimed, 2.0, colmax)

        onehot = (idx_s == idx2).astype(jnp.float32)
        labels_sel = lax.dot_general(lab_ref[s], onehot, dn,
                                     preferred_element_type=jnp.float32)
        labels_v = jnp.where(ovl2 < _THRESHOLD, 0.0, labels_sel)
        positive = labels_v > 0.0
        posf = positive.astype(jnp.float32)

        bsel = lax.dot_general(boxes_t_ref[s], onehot, dn,
                               preferred_element_type=jnp.float32)
        ksel = lax.dot_general(keyp_t_ref[s], onehot, dn,
                               preferred_element_type=jnp.float32)
        dsel = lax.dot_general(dep_t_ref[s], onehot, dn,
                               preferred_element_type=jnp.float32)

        mx1, my1, mx2, my2 = bsel[0:1], bsel[1:2], bsel[2:3], bsel[3:4]
        gcx = ((mx1 + mx2) / 2.0 - pcx) / (_V0 * pw)
        gcy = ((my1 + my2) / 2.0 - pcy) / (_V0 * ph)
        gw = jnp.log(jnp.maximum((mx2 - mx1) / pw, 1e-8)) / _V1
        gh = jnp.log(jnp.maximum((my2 - my1) / ph, 1e-8)) / _V1
        benc = jnp.concatenate([gcx, gcy, gw, gh], axis=0)

        loss_l += jnp.sum(smooth_l1(loc_ref[s], benc) * posf)

        kenc = (ksel - pc_rep) / pd_rep
        loss_landm += jnp.sum(smooth_l1(kenc, landm_ref[s]) * posf)
        loss_dpth += jnp.sum(((dsel - dpt_ref[s]) ** 2) * posf)

        cf = conf_ref[s]
        c0, c1 = cf[0:1], cf[1:2]
        cm = jnp.maximum(c0, c1)
        lse = cm + jnp.log(jnp.exp(c0 - cm) + jnp.exp(c1 - cm))
        pos_rows.append(posf)
        ce_rows.append(lse - jnp.where(positive, c1, c0))
        v_rows.append(jnp.where(positive, 0.0, lse - c0))

    pos_all = jnp.concatenate(pos_rows, axis=0) != 0.0
    ce_all = jnp.concatenate(ce_rows, axis=0)
    v_all = jnp.concatenate(v_rows, axis=0)

    num_pos = jnp.sum(pos_all.astype(jnp.int32), axis=1,
                      keepdims=True)
    num_neg = jnp.minimum(_NEGPOS * num_pos, _P - 1)
    k = jnp.maximum(num_neg, 1)

    bits = lax.bitcast_convert_type(v_all, jnp.int32)
    iota_l = lax.broadcasted_iota(jnp.int32, (1, _P), 1)

    def vbody(_, carry):
        lo, hi = carry
        mid = lo + (hi - lo) // 2
        cnt = jnp.sum((bits >= mid).astype(jnp.int32), axis=1,
                      keepdims=True)
        ge = cnt >= k
        return (jnp.where(ge, mid, lo), jnp.where(ge, hi, mid))

    init = (jnp.zeros((_S, 1), jnp.int32),
            jnp.full((_S, 1), 0x7F800001, jnp.int32))
    t, _hi = lax.fori_loop(0, 31, vbody, init)
    count_gt = jnp.sum((bits > t).astype(jnp.int32), axis=1,
                       keepdims=True)
    need_eq = k - count_gt
    eq = bits == t

    def ibody(_, carry):
        lo2, hi2 = carry
        active = lo2 < hi2
        mid = (lo2 + hi2) // 2
        cnt = jnp.sum((eq & (iota_l < mid)).astype(jnp.int32), axis=1,
                      keepdims=True)
        ge = cnt >= need_eq
        nlo = jnp.where(ge, lo2, mid + 1)
        nhi = jnp.where(ge, mid, hi2)
        return (jnp.where(active, nlo, lo2), jnp.where(active, nhi, hi2))

    init2 = (jnp.zeros((_S, 1), jnp.int32),
             jnp.full((_S, 1), _P, jnp.int32))
    _lo2, m_thr = lax.fori_loop(0, 15, ibody, init2)

    neg_sel = (bits > t) | (eq & (iota_l < m_thr))
    sel = pos_all | (neg_sel & (num_neg > 0))
    loss_c = jnp.sum(jnp.where(sel, ce_all, 0.0))
    num_pos_tot = jnp.sum(num_pos).astype(jnp.float32)

    lane = lax.broadcasted_iota(jnp.int32, (1, 128), 1)
    vec = (jnp.where(lane == 0, loss_l, 0.0)
           + jnp.where(lane == 1, loss_c, 0.0)
           + jnp.where(lane == 2, loss_landm, 0.0)
           + jnp.where(lane == 3, loss_dpth, 0.0)
           + jnp.where(lane == 4, num_pos_tot, 0.0))

    @pl.when(b == 0)
    def _():
        out_ref[...] = jnp.zeros_like(out_ref)

    out_ref[...] += vec


def kernel(loc, conf, landm, dpt, boxes_gt, keypoints_gt, depths_gt, priors, labels_gt):
    loc_t = jnp.transpose(loc, (0, 2, 1))
    conf_t = jnp.transpose(conf, (0, 2, 1))
    landm_t = jnp.transpose(landm, (0, 2, 1))
    dpt_t = jnp.transpose(dpt, (0, 2, 1))
    boxes_t = jnp.transpose(boxes_gt, (0, 2, 1))
    keyp_t = jnp.transpose(keypoints_gt, (0, 2, 1))
    dep_t = jnp.transpose(depths_gt, (0, 2, 1))
    lab_f = labels_gt.astype(jnp.float32)[:, None, :]
    priors_t = priors.T

    grid = (_B // _S,)
    bs = lambda shp: pl.BlockSpec((_S,) + shp, lambda b: (b, 0, 0))
    out = pl.pallas_call(
        _mbl_kernel,
        grid=grid,
        in_specs=[
            bs((4, _P)),
            bs((2, _P)),
            bs((10, _P)),
            bs((2, _P)),
            bs((_NOBJ, 4)),
            bs((4, _NOBJ)),
            bs((10, _NOBJ)),
            bs((2, _NOBJ)),
            bs((1, _NOBJ)),
            pl.BlockSpec((4, _P), lambda b: (0, 0)),
        ],
        out_specs=pl.BlockSpec((1, 128), lambda b: (0, 0)),
        out_shape=jax.ShapeDtypeStruct((1, 128), jnp.float32),
    )(loc_t, conf_t, landm_t, dpt_t,
      boxes_gt, boxes_t, keyp_t, dep_t, lab_f, priors_t)

    r = out[0]
    npos = r[4]
    n = jnp.maximum(npos, 1.0)
    n1 = jnp.maximum(npos * 10.0, 1.0)
    nd = jnp.maximum(npos * 2.0, 1.0)
    return (r[0] / n, r[1] / n, r[2] / n1, r[3] / nd)
